# Initial kernel scaffold; baseline (speedup 1.0000x reference)
import jax
import jax.numpy as jnp
from jax.experimental import pallas as pl


def kernel(pred):
    raise NotImplementedError("write your pallas kernel here")


# speedup vs baseline: 3.2042x; 3.2042x over previous
import jax
import jax.numpy as jnp
from jax import lax
from jax.experimental import pallas as pl
from jax.experimental.pallas import tpu as pltpu

_IMG = 128.0
_CONF = 0.25
_IOU = 0.45
_NC = 9
_NCAND = 300
_CPAD = 384
_E = 20
_N = 20000
_NPAD = 20480
_ROWS = 160
_HI = jax.lax.Precision.HIGHEST


def _fiota(shape, dim):
    return lax.broadcasted_iota(jnp.int32, shape, dim).astype(jnp.float32)


def _body(pred_ref, out_ref, iou_ref):
    p = pred_ref[0]
    cx, cy, w, h, obj = p[0], p[1], p[2], p[3], p[4]

    best = obj * p[5]
    bcls = jnp.zeros_like(best)
    for k in range(1, _NC):
        c = obj * p[5 + k]
        upd = c > best
        bcls = jnp.where(upd, jnp.float32(k), bcls)
        best = jnp.where(upd, c, best)

    x1p = cx - w * 0.5
    y1p = cy - h * 0.5
    x2p = cx + w * 0.5
    y2p = cy + h * 0.5

    scores = jnp.where(best > _CONF, best, -1.0)

    idx2d = (_fiota((_ROWS, 128), 0) * 128.0
             + _fiota((_ROWS, 128), 1))
    lan = _fiota((1, _CPAD), 1)

    def sel_body(t, carry):
        sc, selr, selc, sels = carry
        m = jnp.max(sc)
        pick = jnp.min(jnp.where(sc == m, idx2d, 3.0e7))
        sc = jnp.where(idx2d == pick, -2.0, sc)
        r = jnp.floor(pick / 128.0)
        cc = pick - r * 128.0
        tm = lan == t.astype(jnp.float32)
        selr = jnp.where(tm, r, selr)
        selc = jnp.where(tm, cc, selc)
        sels = jnp.where(tm, m, sels)
        return sc, selr, selc, sels

    z = jnp.zeros((1, _CPAD), jnp.float32)
    _, selr, selc, sels = lax.fori_loop(
        0, _NCAND, sel_body, (scores, z, z, jnp.full((1, _CPAD), -2.0)))

    rowio = _fiota((_ROWS, _CPAD), 0)
    a_t = (rowio == selr).astype(jnp.float32)
    feats = jnp.concatenate([x1p, y1p, x2p, y2p, bcls], axis=1)
    gath = lax.dot_general(feats, a_t, (((0,), (0,)), ((), ())),
                           precision=_HI)
    cio = _fiota((128, _CPAD), 0)
    colm = (cio == selc).astype(jnp.float32)

    def pickf(i):
        return jnp.sum(gath[128 * i:128 * (i + 1), :] * colm,
                       axis=0, keepdims=True)

    x1v, y1v, x2v, y2v, clsv = (pickf(i) for i in range(5))
    confv = sels

    off = clsv * (_IMG * 64.0)
    ox1, oy1, ox2, oy2 = x1v + off, y1v + off, x2v + off, y2v + off
    area = (ox2 - ox1) * (oy2 - oy1)

    subio = _fiota((_CPAD, _CPAD), 0)
    lanio = _fiota((_CPAD, _CPAD), 1)
    ident = (subio == lanio).astype(jnp.float32)

    def tcol(v):
        return lax.dot_general(ident, v, (((1,), (1,)), ((), ())),
                               precision=_HI)

    ox1c, oy1c, ox2c, oy2c, areac = (tcol(v) for v in
                                     (ox1, oy1, ox2, oy2, area))

    xx1 = jnp.maximum(ox1c, ox1)
    yy1 = jnp.maximum(oy1c, oy1)
    xx2 = jnp.minimum(ox2c, ox2)
    yy2 = jnp.minimum(oy2c, oy2)
    iw = jnp.clip(xx2 - xx1, 0.0, None)
    ih = jnp.clip(yy2 - yy1, 0.0, None)
    inter = iw * ih
    iou_ref[...] = inter / (areac + area - inter + 1e-9)

    keep0 = (confv > _CONF).astype(jnp.float32)

    def nms_body(i, keepv):
        ifl = i.astype(jnp.float32)
        row = iou_ref[pl.ds(i, 1), :]
        keep_i = jnp.max(jnp.where(lan == ifl, keepv, 0.0))
        sup = (row > _IOU) & (lan > ifl) & (keep_i > 0.0)
        return jnp.where(sup, 0.0, keepv)

    keepv = lax.fori_loop(0, _NCAND, nms_body, keep0)

    tri = (subio <= lanio).astype(jnp.float32)
    rank = lax.dot_general(keepv, tri, (((1,), (0,)), ((), ())),
                           precision=_HI) - 1.0
    eio = _fiota((_E, _CPAD), 0)
    oneh = ((rank == eio) & (keepv > 0.0)).astype(jnp.float32)

    def pack(v):
        return lax.dot_general(oneh, v, (((1,), (1,)), ((), ())),
                               precision=_HI)

    x1o, y1o, x2o, y2o, clso, probo = (pack(v) for v in
                                       (x1v, y1v, x2v, y2v, clsv, confv))

    cgrp = jnp.floor(clso / 3.0)
    sgrp = clso - 3.0 * cgrp
    cio3 = _fiota((_E, 3), 1)
    color = jnp.where(cgrp == cio3, probo, 0.0)
    shp = jnp.where(sgrp == cio3, probo, 0.0)
    xy = jnp.concatenate([x1o, y1o, x2o, y2o], axis=1) * (1.0 / _IMG)
    out_ref[0] = jnp.concatenate([xy, color, shp, probo], axis=1)


@jax.jit
def kernel(pred):
    b = pred.shape[0]
    pp = jnp.pad(pred, ((0, 0), (0, _NPAD - _N), (0, 0)))
    pp = pp.transpose(0, 2, 1).reshape(b, 14, _ROWS, 128)
    return pl.pallas_call(
        _body,
        grid=(b,),
        in_specs=[pl.BlockSpec((1, 14, _ROWS, 128), lambda i: (i, 0, 0, 0))],
        out_specs=pl.BlockSpec((1, _E, 11), lambda i: (i, 0, 0)),
        out_shape=jax.ShapeDtypeStruct((b, _E, 11), jnp.float32),
        scratch_shapes=[pltpu.VMEM((_CPAD, _CPAD), jnp.float32)],
    )(pp)
